# Initial kernel scaffold; baseline (speedup 1.0000x reference)
import jax
import jax.numpy as jnp
from jax.experimental import pallas as pl


def kernel(x, seg, tok_table, seg_table, gamma, beta, pe):
    raise NotImplementedError("write your pallas kernel here")


# speedup vs baseline: 3.6405x; 3.6405x over previous
import functools

import jax
import jax.numpy as jnp
from jax import lax
from jax.experimental import pallas as pl
from jax.experimental.pallas import tpu as pltpu
from jax.experimental.pallas import tpu_sc as plsc

VOCAB = 100000
D = 128
B = 1024
L = 512
N = B * L

NC = 2
NS = 16
NW = NC * NS

ROWS_PER_W = N // NW
CHUNK = 128
NCHUNK = ROWS_PER_W // CHUNK


def _sc_gather_body(x_hbm, table_hbm, out_hbm, idx_v, rows_v, sem):
  wid = lax.axis_index("s") * NC + lax.axis_index("c")
  base_w = wid * ROWS_PER_W

  def chunk_body(c, _):
    base = base_w + c * CHUNK
    pltpu.sync_copy(x_hbm.at[pl.ds(base, CHUNK)], idx_v)
    pltpu.async_copy(table_hbm.at[idx_v], rows_v, sem).wait()
    pltpu.sync_copy(rows_v, out_hbm.at[pl.ds(base, CHUNK)])
    return _

  lax.fori_loop(0, NCHUNK, chunk_body, None, unroll=False)


@jax.jit
def _sc_gather(xf, table):
  mesh = plsc.VectorSubcoreMesh(core_axis_name="c", subcore_axis_name="s")
  return pl.kernel(
      _sc_gather_body,
      out_type=jax.ShapeDtypeStruct((N, D), jnp.float32),
      mesh=mesh,
      scratch_types=[
          pltpu.VMEM((CHUNK,), jnp.int32),
          pltpu.VMEM((CHUNK, D), jnp.float32),
          pltpu.SemaphoreType.DMA,
      ],
  )(xf, table)


def _ln_body(tok_ref, seg_ref, pe_ref, segtab_ref, gamma_ref, beta_ref, o_ref):
  h = tok_ref[...] + pe_ref[...]
  s = seg_ref[0, 0, :][:, None]
  segtab = segtab_ref[...]
  seg_emb = jnp.where(s == 0, segtab[0][None, :],
                      jnp.where(s == 1, segtab[1][None, :],
                                segtab[2][None, :]))
  h = h + seg_emb
  mean = jnp.mean(h, axis=-1, keepdims=True)
  var = jnp.mean(jnp.square(h - mean), axis=-1, keepdims=True)
  inv = lax.rsqrt(var + 1e-5)
  o_ref[...] = (h - mean) * inv * gamma_ref[...] + beta_ref[...]


@jax.jit
def _tc_ln(tok_rows, seg3d, pe2d, seg_table, gamma, beta):
  grid = (B,)
  return pl.pallas_call(
      _ln_body,
      grid=grid,
      in_specs=[
          pl.BlockSpec((L, D), lambda i: (i, 0)),
          pl.BlockSpec((1, 1, L), lambda i: (i, 0, 0)),
          pl.BlockSpec((L, D), lambda i: (0, 0)),
          pl.BlockSpec((3, D), lambda i: (0, 0)),
          pl.BlockSpec((1, D), lambda i: (0, 0)),
          pl.BlockSpec((1, D), lambda i: (0, 0)),
      ],
      out_specs=pl.BlockSpec((L, D), lambda i: (i, 0)),
      out_shape=jax.ShapeDtypeStruct((N, D), jnp.float32),
  )(tok_rows, seg3d, pe2d, seg_table, gamma, beta)


def kernel(x, seg, tok_table, seg_table, gamma, beta, pe):
  xf = x.reshape(-1)
  seg3d = seg.reshape(B, 1, L)
  pe2d = pe.reshape(pe.shape[1], D)[:L]
  tok_rows = _sc_gather(xf, tok_table)
  out = _tc_ln(tok_rows, seg3d, pe2d, seg_table,
               gamma.reshape(1, D), beta.reshape(1, D))
  return out.reshape(B, L, D)
